# baseline (speedup 1.0000x reference)
import jax
import jax.numpy as jnp
from jax import lax
from jax.experimental import pallas as pl
from jax.experimental.pallas import tpu as pltpu
from jax.experimental.pallas import tpu_sc as plsc

_DIM = 64
_CHUNK = 128
_NBUF = 2
_SCALE = 8.0
_L = 16
_NW = 32
_GROUPS = _CHUNK // _L


def _sc_embed(tok_flat, tab2):
    n = tok_flat.shape[0]
    per_w = n // _NW
    nchunk = per_w // _CHUNK
    ngroups = per_w // _L
    mesh = plsc.VectorSubcoreMesh(core_axis_name="c", subcore_axis_name="s")

    @pl.kernel(
        out_type=jax.ShapeDtypeStruct((n, _DIM), jnp.float32),
        mesh=mesh,
        compiler_params=pltpu.CompilerParams(needs_layout_passes=False),
        scratch_types=[
            pltpu.VMEM((per_w,), jnp.int32),
            pltpu.VMEM((per_w,), jnp.int32),
            pltpu.VMEM((per_w,), jnp.int32),
            pltpu.VMEM((_NBUF, _CHUNK, 2 * _DIM), jnp.float32),
            pltpu.VMEM((_NBUF, _CHUNK, _DIM), jnp.float32),
            pltpu.SemaphoreType.DMA,
            pltpu.SemaphoreType.DMA,
            pltpu.SemaphoreType.DMA,
        ],
    )
    def k(tab_hbm, tok_hbm, out_hbm, tok_v, pair_v, half_v, gbuf, obuf,
          sem_i, sem_g, sem_o):
        wid = lax.axis_index("s") * 2 + lax.axis_index("c")
        base = wid * per_w
        pltpu.async_copy(tok_hbm.at[pl.ds(base, per_w)], tok_v, sem_i).wait()

        iota = lax.iota(jnp.int32, _L)

        @pl.loop(0, ngroups)
        def _(g):
            t16 = tok_v.at[pl.ds(g * _L, _L)][...]
            pair_v.at[pl.ds(g * _L, _L)][...] = t16 >> 1
            half_v.at[pl.ds(g * _L, _L)][...] = (t16 & 1) << 6

        def gather(kk, b):
            return pltpu.make_async_copy(
                tab_hbm.at[pair_v.at[pl.ds(kk * _CHUNK, _CHUNK)]],
                gbuf.at[b],
                sem_g,
            )

        def put(kk, b):
            return pltpu.make_async_copy(
                obuf.at[b],
                out_hbm.at[pl.ds(base + kk * _CHUNK, _CHUNK)],
                sem_o,
            )

        for b in range(_NBUF):
            gather(b, b).start()

        @pl.loop(0, nchunk, step=_NBUF)
        def _(k0):
            for b in range(_NBUF):
                kk = k0 + b
                gather(kk, b).wait()

                @pl.when(kk >= _NBUF)
                def _():
                    put(kk - _NBUF, b).wait()

                for g in range(_GROUPS):
                    row16 = iota + (g * _L)
                    half16 = half_v.at[pl.ds(kk * _CHUNK + g * _L, _L)][...]

                    @pl.loop(0, _DIM)
                    def _(cc):
                        cc16 = jnp.full((_L,), cc, jnp.int32)
                        vals = plsc.load_gather(
                            gbuf.at[b], [row16, half16 + cc16]
                        )
                        plsc.store_scatter(
                            obuf.at[b], [row16, cc16], vals * _SCALE
                        )

                put(kk, b).start()

                @pl.when(kk + _NBUF < nchunk)
                def _():
                    gather(kk + _NBUF, b).start()

        for b in range(_NBUF):
            put(nchunk - _NBUF + b, b).wait()

    return k(tab2, tok_flat)


def kernel(tokens, table):
    b, s = tokens.shape
    tok_flat = tokens.astype(jnp.int32).reshape(b * s)
    tab2 = table.reshape(table.shape[0] // 2, 2 * _DIM)
    out = _sc_embed(tok_flat, tab2)
    return out.reshape(b, s, _DIM)


# speedup vs baseline: 1.8198x; 1.8198x over previous
import jax
import jax.numpy as jnp
from jax import lax
from jax.experimental import pallas as pl
from jax.experimental.pallas import tpu as pltpu
from jax.experimental.pallas import tpu_sc as plsc

_DIM = 64
_PAD = 128
_BCHUNK = 4
_NBUF = 2
_SCALE = 8.0
_L = 16
_NW = 32


def _sc_embed(tokens, tab_pad):
    nb, ns = tokens.shape
    n = nb * ns
    per_w = n // _NW
    tchunk = 128
    nchunk = per_w // tchunk
    mesh = plsc.VectorSubcoreMesh(core_axis_name="c", subcore_axis_name="s")

    @pl.kernel(
        out_type=jax.ShapeDtypeStruct((n, _DIM), jnp.float32),
        mesh=mesh,
        compiler_params=pltpu.CompilerParams(needs_layout_passes=False),
        scratch_types=[
            pltpu.VMEM((per_w,), jnp.int32),
            pltpu.VMEM((_NBUF, tchunk, _PAD), jnp.float32),
            pltpu.VMEM((_NBUF, tchunk, _DIM), jnp.float32),
            pltpu.SemaphoreType.DMA,
            pltpu.SemaphoreType.DMA,
            pltpu.SemaphoreType.DMA,
        ],
    )
    def k(tab_hbm, tok_hbm, out_hbm, idx_v, gbuf, obuf, sem_i, sem_g, sem_o):
        wid = lax.axis_index("s") * 2 + lax.axis_index("c")
        base = wid * per_w
        pltpu.async_copy(tok_hbm.at[pl.ds(base, per_w)], idx_v, sem_i).wait()

        def gather(kk, b):
            return pltpu.make_async_copy(
                tab_hbm.at[idx_v.at[pl.ds(kk * tchunk, tchunk)]],
                gbuf.at[b],
                sem_g,
            )

        def put(kk, b):
            return pltpu.make_async_copy(
                obuf.at[b],
                out_hbm.at[pl.ds(base + kk * tchunk, tchunk)],
                sem_o,
            )

        for b in range(_NBUF):
            gather(b, b).start()

        @pl.loop(0, nchunk, step=_NBUF)
        def _(k0):
            for b in range(_NBUF):
                kk = k0 + b
                gather(kk, b).wait()

                @pl.when(kk >= _NBUF)
                def _():
                    put(kk - _NBUF, b).wait()

                @pl.loop(0, tchunk)
                def _(rr):
                    for c in range(0, _DIM, _L):
                        obuf.at[b, rr, pl.ds(c, _L)][...] = (
                            gbuf.at[b, rr, pl.ds(c, _L)][...] * _SCALE
                        )

                put(kk, b).start()

                @pl.when(kk + _NBUF < nchunk)
                def _():
                    gather(kk + _NBUF, b).start()

        for b in range(_NBUF):
            put(nchunk - _NBUF + b, b).wait()

    out = k(tab_pad, tokens.astype(jnp.int32).reshape(n))
    return out.reshape(nb, ns, _DIM)


def kernel(tokens, table):
    tab_pad = jnp.pad(table, ((0, 0), (0, _PAD - _DIM)))
    return _sc_embed(tokens, tab_pad)
